# baseline (speedup 1.0000x reference)
import functools

import jax
import jax.numpy as jnp
from jax import lax
from jax.experimental import pallas as pl
from jax.experimental.pallas import tpu as pltpu
from jax.experimental.pallas import tpu_sc as plsc

WIDTH = 512
STEP = 256
LANES = 16


SC_G = 8
SC_IG = 32
SC_SUB = SC_IG // SC_G


def _sc_body(x_hbm, out_hbm, win_v0, win_v1, out_v0, out_v1, carry_v,
             isem0, isem1, osem0, osem1, *, n, ng):
    ch = lax.axis_index("s") * 2 + lax.axis_index("c")
    wins = (win_v0, win_v1)
    outs = (out_v0, out_v1)
    isems = (isem0, isem1)
    osems = (osem0, osem1)

    def in_cp(g, b):
        return pltpu.make_async_copy(
            x_hbm.at[pl.ds(g * SC_IG, SC_IG), pl.ds(ch, 1), :], wins[b], isems[b])

    def out_cp(g, b):
        return pltpu.make_async_copy(
            outs[b], out_hbm.at[pl.ds(ch, 1), pl.ds(g * SC_IG * STEP, SC_IG * STEP)],
            osems[b])

    zero = jnp.zeros((LANES,), jnp.float32)
    for i in range(STEP // LANES):
        carry_v[0, pl.ds(i * LANES, LANES)] = zero

    in_cp(0, 0).start()

    def outer(g2, _):
        for b in range(2):
            g = g2 * 2 + b

            @pl.when(g + 1 < ng)
            def _():
                in_cp(g + 1, 1 - b).start()

            in_cp(g, b).wait()

            @pl.when(g >= 2)
            def _():
                out_cp(g - 2, b).wait()

            win_v = wins[b]
            out_v = outs[b]

            def sub(s, _):
                base = s * SC_G
                for k in range(SC_G):
                    for i in range(STEP // LANES):
                        off = i * LANES
                        a = win_v[base + k, 0, pl.ds(off, LANES)]
                        if k == 0:
                            b_half = carry_v[0, pl.ds(off, LANES)]
                        else:
                            b_half = win_v[base + k - 1, 0, pl.ds(STEP + off, LANES)]
                        out_v[0, pl.ds(base * STEP + k * STEP + off, LANES)] = a + b_half
                for i in range(STEP // LANES):
                    off = i * LANES
                    carry_v[0, pl.ds(off, LANES)] = (
                        win_v[base + SC_G - 1, 0, pl.ds(STEP + off, LANES)])
                return 0

            lax.fori_loop(0, SC_SUB, sub, 0)
            out_cp(g, b).start()
        return 0

    lax.fori_loop(0, ng // 2, outer, 0)
    for b in range(2):
        out_cp(ng - 2 + b, b).wait()
    pltpu.sync_copy(carry_v, out_hbm.at[pl.ds(ch, 1), pl.ds(n * STEP, STEP)])


def _sc_kernel(x):
    n, c, w = x.shape
    total = (n - 1) * STEP + w
    ng = n // SC_IG
    mesh = plsc.VectorSubcoreMesh(core_axis_name="c", subcore_axis_name="s")
    kfn = pl.kernel(
        functools.partial(_sc_body, n=n, ng=ng),
        out_type=jax.ShapeDtypeStruct((c, total), x.dtype),
        mesh=mesh,
        scratch_types=[
            pltpu.VMEM((SC_IG, 1, WIDTH), jnp.float32),
            pltpu.VMEM((SC_IG, 1, WIDTH), jnp.float32),
            pltpu.VMEM((1, SC_IG * STEP), jnp.float32),
            pltpu.VMEM((1, SC_IG * STEP), jnp.float32),
            pltpu.VMEM((1, STEP), jnp.float32),
            pltpu.SemaphoreType.DMA,
            pltpu.SemaphoreType.DMA,
            pltpu.SemaphoreType.DMA,
            pltpu.SemaphoreType.DMA,
        ],
    )
    return kfn(x)


G = 8


def _tc_body(x_ref, o_ref, carry_ref, *, nb):
    j = pl.program_id(0)

    @pl.when(j < nb)
    def _main():
        a0 = x_ref[0, :, :STEP]
        o_ref[:, :STEP] = jnp.where(j == 0, a0, a0 + carry_ref[...])
        for k in range(1, G):
            o_ref[:, k * STEP:(k + 1) * STEP] = (
                x_ref[k, :, :STEP] + x_ref[k - 1, :, STEP:])
        carry_ref[...] = x_ref[G - 1, :, STEP:]

    @pl.when(j == nb)
    def _tail():
        o_ref[:, :STEP] = carry_ref[...]


def _tc_kernel(x):
    n, c, w = x.shape
    total = (n - 1) * STEP + w
    nb = n // G
    return pl.pallas_call(
        functools.partial(_tc_body, nb=nb),
        grid=(nb + 1,),
        in_specs=[pl.BlockSpec((G, c, w), lambda j: (jnp.minimum(j, nb - 1), 0, 0))],
        out_specs=pl.BlockSpec((c, G * STEP), lambda j: (0, j)),
        out_shape=jax.ShapeDtypeStruct((c, total), x.dtype),
        scratch_shapes=[pltpu.VMEM((c, STEP), x.dtype)],
    )(x)


def kernel(input_time_series):
    return _sc_kernel(input_time_series)


# speedup vs baseline: 1.1707x; 1.1707x over previous
import functools

import jax
import jax.numpy as jnp
from jax import lax
from jax.experimental import pallas as pl
from jax.experimental.pallas import tpu as pltpu
from jax.experimental.pallas import tpu_sc as plsc

WIDTH = 512
STEP = 256
LANES = 16


SC_G = 8
SC_IG = 32
SC_SUB = SC_IG // SC_G


def _sc_body(x_hbm, out_hbm, win_v0, win_v1, out_v0, out_v1, carry_v,
             isem0, isem1, osem0, osem1, *, n, ng):
    ch = lax.axis_index("s") * 2 + lax.axis_index("c")
    wins = (win_v0, win_v1)
    outs = (out_v0, out_v1)
    isems = (isem0, isem1)
    osems = (osem0, osem1)

    def in_cp(g, b):
        return pltpu.make_async_copy(
            x_hbm.at[pl.ds(g * SC_IG, SC_IG), pl.ds(ch, 1), :], wins[b], isems[b])

    def out_cp(g, b):
        return pltpu.make_async_copy(
            outs[b], out_hbm.at[pl.ds(ch, 1), pl.ds(g * SC_IG * STEP, SC_IG * STEP)],
            osems[b])

    zero = jnp.zeros((LANES,), jnp.float32)
    for i in range(STEP // LANES):
        carry_v[0, pl.ds(i * LANES, LANES)] = zero

    in_cp(0, 0).start()

    def outer(g2, _):
        for b in range(2):
            g = g2 * 2 + b

            @pl.when(g + 1 < ng)
            def _():
                in_cp(g + 1, 1 - b).start()

            in_cp(g, b).wait()

            @pl.when(g >= 2)
            def _():
                out_cp(g - 2, b).wait()

            win_v = wins[b]
            out_v = outs[b]

            def sub(s):
                base = s * SC_G
                for k in range(SC_G):
                    for i in range(STEP // LANES):
                        off = i * LANES
                        a = win_v[base + k, 0, pl.ds(off, LANES)]
                        if k == 0:
                            b_half = carry_v[0, pl.ds(off, LANES)]
                        else:
                            b_half = win_v[base + k - 1, 0, pl.ds(STEP + off, LANES)]
                        out_v[0, pl.ds(base * STEP + k * STEP + off, LANES)] = a + b_half
                for i in range(STEP // LANES):
                    off = i * LANES
                    carry_v[0, pl.ds(off, LANES)] = (
                        win_v[base + SC_G - 1, 0, pl.ds(STEP + off, LANES)])

            for s in range(SC_SUB):
                sub(s)
            out_cp(g, b).start()
        return 0

    lax.fori_loop(0, ng // 2, outer, 0)
    for b in range(2):
        out_cp(ng - 2 + b, b).wait()
    pltpu.sync_copy(carry_v, out_hbm.at[pl.ds(ch, 1), pl.ds(n * STEP, STEP)])


def _sc_kernel(x):
    n, c, w = x.shape
    total = (n - 1) * STEP + w
    ng = n // SC_IG
    mesh = plsc.VectorSubcoreMesh(core_axis_name="c", subcore_axis_name="s")
    kfn = pl.kernel(
        functools.partial(_sc_body, n=n, ng=ng),
        out_type=jax.ShapeDtypeStruct((c, total), x.dtype),
        mesh=mesh,
        scratch_types=[
            pltpu.VMEM((SC_IG, 1, WIDTH), jnp.float32),
            pltpu.VMEM((SC_IG, 1, WIDTH), jnp.float32),
            pltpu.VMEM((1, SC_IG * STEP), jnp.float32),
            pltpu.VMEM((1, SC_IG * STEP), jnp.float32),
            pltpu.VMEM((1, STEP), jnp.float32),
            pltpu.SemaphoreType.DMA,
            pltpu.SemaphoreType.DMA,
            pltpu.SemaphoreType.DMA,
            pltpu.SemaphoreType.DMA,
        ],
    )
    return kfn(x)


G = 8


def _tc_body(x_ref, o_ref, carry_ref, *, nb):
    j = pl.program_id(0)

    @pl.when(j < nb)
    def _main():
        a0 = x_ref[0, :, :STEP]
        o_ref[:, :STEP] = jnp.where(j == 0, a0, a0 + carry_ref[...])
        for k in range(1, G):
            o_ref[:, k * STEP:(k + 1) * STEP] = (
                x_ref[k, :, :STEP] + x_ref[k - 1, :, STEP:])
        carry_ref[...] = x_ref[G - 1, :, STEP:]

    @pl.when(j == nb)
    def _tail():
        o_ref[:, :STEP] = carry_ref[...]


def _tc_kernel(x):
    n, c, w = x.shape
    total = (n - 1) * STEP + w
    nb = n // G
    return pl.pallas_call(
        functools.partial(_tc_body, nb=nb),
        grid=(nb + 1,),
        in_specs=[pl.BlockSpec((G, c, w), lambda j: (jnp.minimum(j, nb - 1), 0, 0))],
        out_specs=pl.BlockSpec((c, G * STEP), lambda j: (0, j)),
        out_shape=jax.ShapeDtypeStruct((c, total), x.dtype),
        scratch_shapes=[pltpu.VMEM((c, STEP), x.dtype)],
    )(x)


def kernel(input_time_series):
    return _sc_kernel(input_time_series)


# speedup vs baseline: 1.4726x; 1.2579x over previous
import functools

import jax
import jax.numpy as jnp
from jax import lax
from jax.experimental import pallas as pl
from jax.experimental.pallas import tpu as pltpu
from jax.experimental.pallas import tpu_sc as plsc

WIDTH = 512
STEP = 256
LANES = 16


SC_G = 8
SC_NBUF = 4


def _sc_body(x_hbm, out_hbm, *refs, n, ng):
    wins = refs[0:SC_NBUF]
    outs = refs[SC_NBUF:2 * SC_NBUF]
    carry_v = refs[2 * SC_NBUF]
    isems = refs[2 * SC_NBUF + 1:3 * SC_NBUF + 1]
    osems = refs[3 * SC_NBUF + 1:4 * SC_NBUF + 1]
    ch = lax.axis_index("s") * 2 + lax.axis_index("c")

    def in_cp(g, b):
        return pltpu.make_async_copy(
            x_hbm.at[pl.ds(g * SC_G, SC_G), pl.ds(ch, 1), :], wins[b], isems[b])

    def out_cp(g, b):
        return pltpu.make_async_copy(
            outs[b], out_hbm.at[pl.ds(ch, 1), pl.ds(g * SC_G * STEP, SC_G * STEP)],
            osems[b])

    zero = jnp.zeros((LANES,), jnp.float32)
    for i in range(STEP // LANES):
        carry_v[0, pl.ds(i * LANES, LANES)] = zero

    for b in range(SC_NBUF - 1):
        in_cp(b, b).start()

    def outer(gq, _):
        for b in range(SC_NBUF):
            g = gq * SC_NBUF + b

            @pl.when(g + SC_NBUF - 1 < ng)
            def _():
                in_cp(g + SC_NBUF - 1, (b + SC_NBUF - 1) % SC_NBUF).start()

            in_cp(g, b).wait()

            @pl.when(g >= SC_NBUF)
            def _():
                out_cp(g - SC_NBUF, b).wait()

            win_v = wins[b]
            out_v = outs[b]
            for k in range(SC_G):
                for i in range(STEP // LANES):
                    off = i * LANES
                    a = win_v[k, 0, pl.ds(off, LANES)]
                    if k == 0:
                        b_half = carry_v[0, pl.ds(off, LANES)]
                    else:
                        b_half = win_v[k - 1, 0, pl.ds(STEP + off, LANES)]
                    out_v[0, pl.ds(k * STEP + off, LANES)] = a + b_half
            for i in range(STEP // LANES):
                off = i * LANES
                carry_v[0, pl.ds(off, LANES)] = (
                    win_v[SC_G - 1, 0, pl.ds(STEP + off, LANES)])
            out_cp(g, b).start()
        return 0

    lax.fori_loop(0, ng // SC_NBUF, outer, 0)
    for b in range(SC_NBUF):
        out_cp(ng - SC_NBUF + b, b).wait()
    pltpu.sync_copy(carry_v, out_hbm.at[pl.ds(ch, 1), pl.ds(n * STEP, STEP)])


def _sc_kernel(x):
    n, c, w = x.shape
    total = (n - 1) * STEP + w
    ng = n // SC_G
    mesh = plsc.VectorSubcoreMesh(core_axis_name="c", subcore_axis_name="s")
    kfn = pl.kernel(
        functools.partial(_sc_body, n=n, ng=ng),
        out_type=jax.ShapeDtypeStruct((c, total), x.dtype),
        mesh=mesh,
        scratch_types=(
            [pltpu.VMEM((SC_G, 1, WIDTH), jnp.float32)] * SC_NBUF
            + [pltpu.VMEM((1, SC_G * STEP), jnp.float32)] * SC_NBUF
            + [pltpu.VMEM((1, STEP), jnp.float32)]
            + [pltpu.SemaphoreType.DMA] * (2 * SC_NBUF)
        ),
    )
    return kfn(x)


G = 8


def _tc_body(x_ref, o_ref, carry_ref, *, nb):
    j = pl.program_id(0)

    @pl.when(j < nb)
    def _main():
        a0 = x_ref[0, :, :STEP]
        o_ref[:, :STEP] = jnp.where(j == 0, a0, a0 + carry_ref[...])
        for k in range(1, G):
            o_ref[:, k * STEP:(k + 1) * STEP] = (
                x_ref[k, :, :STEP] + x_ref[k - 1, :, STEP:])
        carry_ref[...] = x_ref[G - 1, :, STEP:]

    @pl.when(j == nb)
    def _tail():
        o_ref[:, :STEP] = carry_ref[...]


def _tc_kernel(x):
    n, c, w = x.shape
    total = (n - 1) * STEP + w
    nb = n // G
    return pl.pallas_call(
        functools.partial(_tc_body, nb=nb),
        grid=(nb + 1,),
        in_specs=[pl.BlockSpec((G, c, w), lambda j: (jnp.minimum(j, nb - 1), 0, 0))],
        out_specs=pl.BlockSpec((c, G * STEP), lambda j: (0, j)),
        out_shape=jax.ShapeDtypeStruct((c, total), x.dtype),
        scratch_shapes=[pltpu.VMEM((c, STEP), x.dtype)],
    )(x)


def kernel(input_time_series):
    return _sc_kernel(input_time_series)


# speedup vs baseline: 1.6954x; 1.1513x over previous
import functools

import jax
import jax.numpy as jnp
from jax import lax
from jax.experimental import pallas as pl
from jax.experimental.pallas import tpu as pltpu
from jax.experimental.pallas import tpu_sc as plsc

WIDTH = 512
STEP = 256
LANES = 16


SC_G = 8
SC_NBUF = 4
SC_LOOK = 2
SC_OWAIT = 2


def _sc_body(x_hbm, out_hbm, *refs, n, ng):
    wins = refs[0:SC_NBUF]
    outs = refs[SC_NBUF:2 * SC_NBUF]
    carry_v = refs[2 * SC_NBUF]
    isems = refs[2 * SC_NBUF + 1:3 * SC_NBUF + 1]
    osems = refs[3 * SC_NBUF + 1:4 * SC_NBUF + 1]
    ch = lax.axis_index("s") * 2 + lax.axis_index("c")

    def in_cp(g, b):
        return pltpu.make_async_copy(
            x_hbm.at[pl.ds(g * SC_G, SC_G), pl.ds(ch, 1), :], wins[b], isems[b])

    def out_cp(g, b):
        return pltpu.make_async_copy(
            outs[b], out_hbm.at[pl.ds(ch, 1), pl.ds(g * SC_G * STEP, SC_G * STEP)],
            osems[b])

    zero = jnp.zeros((LANES,), jnp.float32)
    for i in range(STEP // LANES):
        carry_v[0, pl.ds(i * LANES, LANES)] = zero

    for d in range(SC_LOOK):
        in_cp(d, d).start()

    def outer(gq, _):
        for b in range(SC_NBUF):
            g = gq * SC_NBUF + b

            @pl.when(g + SC_LOOK < ng)
            def _():
                in_cp(g + SC_LOOK, (b + SC_LOOK) % SC_NBUF).start()

            in_cp(g, b).wait()

            @pl.when(g >= SC_OWAIT)
            def _():
                out_cp(g - SC_OWAIT, (b - SC_OWAIT) % SC_NBUF).wait()

            win_v = wins[b]
            out_v = outs[b]
            for k in range(SC_G):
                for i in range(STEP // LANES):
                    off = i * LANES
                    a = win_v[k, 0, pl.ds(off, LANES)]
                    if k == 0:
                        b_half = carry_v[0, pl.ds(off, LANES)]
                    else:
                        b_half = win_v[k - 1, 0, pl.ds(STEP + off, LANES)]
                    out_v[0, pl.ds(k * STEP + off, LANES)] = a + b_half
            for i in range(STEP // LANES):
                off = i * LANES
                carry_v[0, pl.ds(off, LANES)] = (
                    win_v[SC_G - 1, 0, pl.ds(STEP + off, LANES)])
            out_cp(g, b).start()
        return 0

    lax.fori_loop(0, ng // SC_NBUF, outer, 0)
    for d in range(SC_OWAIT):
        g = ng - SC_OWAIT + d
        out_cp(g, g % SC_NBUF).wait()
    pltpu.sync_copy(carry_v, out_hbm.at[pl.ds(ch, 1), pl.ds(n * STEP, STEP)])


def _sc_kernel(x):
    n, c, w = x.shape
    total = (n - 1) * STEP + w
    ng = n // SC_G
    mesh = plsc.VectorSubcoreMesh(core_axis_name="c", subcore_axis_name="s")
    kfn = pl.kernel(
        functools.partial(_sc_body, n=n, ng=ng),
        out_type=jax.ShapeDtypeStruct((c, total), x.dtype),
        mesh=mesh,
        scratch_types=(
            [pltpu.VMEM((SC_G, 1, WIDTH), jnp.float32)] * SC_NBUF
            + [pltpu.VMEM((1, SC_G * STEP), jnp.float32)] * SC_NBUF
            + [pltpu.VMEM((1, STEP), jnp.float32)]
            + [pltpu.SemaphoreType.DMA] * (2 * SC_NBUF)
        ),
    )
    return kfn(x)


G = 8


def _tc_body(x_ref, o_ref, carry_ref, *, nb):
    j = pl.program_id(0)

    @pl.when(j < nb)
    def _main():
        a0 = x_ref[0, :, :STEP]
        o_ref[:, :STEP] = jnp.where(j == 0, a0, a0 + carry_ref[...])
        for k in range(1, G):
            o_ref[:, k * STEP:(k + 1) * STEP] = (
                x_ref[k, :, :STEP] + x_ref[k - 1, :, STEP:])
        carry_ref[...] = x_ref[G - 1, :, STEP:]

    @pl.when(j == nb)
    def _tail():
        o_ref[:, :STEP] = carry_ref[...]


def _tc_kernel(x):
    n, c, w = x.shape
    total = (n - 1) * STEP + w
    nb = n // G
    return pl.pallas_call(
        functools.partial(_tc_body, nb=nb),
        grid=(nb + 1,),
        in_specs=[pl.BlockSpec((G, c, w), lambda j: (jnp.minimum(j, nb - 1), 0, 0))],
        out_specs=pl.BlockSpec((c, G * STEP), lambda j: (0, j)),
        out_shape=jax.ShapeDtypeStruct((c, total), x.dtype),
        scratch_shapes=[pltpu.VMEM((c, STEP), x.dtype)],
    )(x)


def kernel(input_time_series):
    return _sc_kernel(input_time_series)
